# Initial kernel scaffold; baseline (speedup 1.0000x reference)
import jax
import jax.numpy as jnp
from jax.experimental import pallas as pl


def kernel(features, key_locs):
    raise NotImplementedError("write your pallas kernel here")


# speedup vs baseline: 5.1047x; 5.1047x over previous
import functools

import jax
import jax.numpy as jnp
from jax import lax
from jax.experimental import pallas as pl
from jax.experimental.pallas import tpu as pltpu
from jax.experimental.pallas import tpu_sc as plsc

B = 16
N = 16384
D = 128
HW = 256

NODES_PER_TILE = N // 2
CHUNK = 128
NCHUNK = NODES_PER_TILE // CHUNK
ROWS_PER_SC = 8 * HW
ROWS_PER_TILE = ROWS_PER_SC // 16


def _sc_body(feat_hbm, ky_hbm, kx_hbm, out_hbm,
             ybuf, xbuf, cells, feat, ones, acc, cnt, sums_t, cnt_t):
    c = lax.axis_index("c")
    s = lax.axis_index("s")

    zeros16f = jnp.zeros((16,), jnp.float32)
    ones16f = jnp.ones((16,), jnp.float32)

    def _zero_row(i, _):
        for j in range(D // 16):
            sums_t[i, pl.ds(j * 16, 16)] = zeros16f
        for j in range(D // 16):
            cnt_t[i, pl.ds(j * 16, 16)] = zeros16f
            ones[i, pl.ds(j * 16, 16)] = ones16f
        return 0
    lax.fori_loop(0, ROWS_PER_TILE, _zero_row, 0)

    pltpu.sync_copy(sums_t, acc.at[pl.ds(s * ROWS_PER_TILE, ROWS_PER_TILE)])
    pltpu.sync_copy(cnt_t, cnt.at[pl.ds(s * ROWS_PER_TILE, ROWS_PER_TILE)])
    plsc.subcore_barrier()

    lb = s // 2
    b = c * 8 + lb
    half = s % 2
    n0 = half * NODES_PER_TILE

    pltpu.sync_copy(ky_hbm.at[b, pl.ds(n0, NODES_PER_TILE)], ybuf)
    pltpu.sync_copy(kx_hbm.at[b, pl.ds(n0, NODES_PER_TILE)], xbuf)

    row_base = lb * HW

    def _chunk(i, _):
        nstart = i * CHUNK
        pltpu.sync_copy(feat_hbm.at[b, pl.ds(n0 + nstart, CHUNK)], feat)
        for j in range(CHUNK // 16):
            yv = ybuf[pl.ds(nstart + j * 16, 16)]
            xv = xbuf[pl.ds(nstart + j * 16, 16)]
            cells[pl.ds(j * 16, 16)] = yv * 16 + xv + row_base
        pltpu.sync_copy(feat, acc.at[cells], add=True)
        pltpu.sync_copy(ones, cnt.at[cells], add=True)
        return 0

    lax.fori_loop(0, NCHUNK, _chunk, 0)
    plsc.subcore_barrier()

    rows0 = s * ROWS_PER_TILE
    pltpu.sync_copy(acc.at[pl.ds(rows0, ROWS_PER_TILE)], sums_t)
    pltpu.sync_copy(cnt.at[pl.ds(rows0, ROWS_PER_TILE)], cnt_t)

    def _cell(i, _):
        inv = 1.0 / jnp.maximum(cnt_t[i, pl.ds(0, 16)], 1.0)
        for j in range(D // 16):
            sums_t[i, pl.ds(j * 16, 16)] = sums_t[i, pl.ds(j * 16, 16)] * inv
        return 0
    lax.fori_loop(0, ROWS_PER_TILE, _cell, 0)

    b2 = c * 8 + s // 2
    cell0 = (s % 2) * ROWS_PER_TILE
    pltpu.sync_copy(sums_t, out_hbm.at[b2, pl.ds(cell0, ROWS_PER_TILE)])


@jax.jit
def _graph_to_cnn(features, ky, kx):
    mesh = plsc.VectorSubcoreMesh(core_axis_name="c", subcore_axis_name="s")
    run = functools.partial(
        pl.kernel,
        out_type=jax.ShapeDtypeStruct((B, HW, D), jnp.float32),
        mesh=mesh,
        scratch_types=[
            pltpu.VMEM((NODES_PER_TILE,), jnp.int32),
            pltpu.VMEM((NODES_PER_TILE,), jnp.int32),
            pltpu.VMEM((CHUNK,), jnp.int32),
            pltpu.VMEM((CHUNK, D), jnp.float32),
            pltpu.VMEM((CHUNK, D), jnp.float32),
            pltpu.VMEM_SHARED((ROWS_PER_SC, D), jnp.float32),
            pltpu.VMEM_SHARED((ROWS_PER_SC, D), jnp.float32),
            pltpu.VMEM((ROWS_PER_TILE, D), jnp.float32),
            pltpu.VMEM((ROWS_PER_TILE, D), jnp.float32),
        ],
    )(_sc_body)
    return run(features, ky, kx)


def kernel(features, key_locs):
    kl = key_locs.astype(jnp.int32)
    ky = kl[..., 0]
    kx = kl[..., 1]
    out = _graph_to_cnn(features, ky, kx)
    return jnp.transpose(out, (0, 2, 1)).reshape(B, D, 16, 16)


# speedup vs baseline: 7.1875x; 1.4080x over previous
import functools

import jax
import jax.numpy as jnp
from jax import lax
from jax.experimental import pallas as pl
from jax.experimental.pallas import tpu as pltpu
from jax.experimental.pallas import tpu_sc as plsc

B = 16
N = 16384
D = 128
HW = 256

NODES_PER_TILE = N // 2
CHUNK = 128
NCHUNK = NODES_PER_TILE // CHUNK
ROWS_PER_SC = 8 * HW
ROWS_PER_TILE = ROWS_PER_SC // 16
NBUF = 4


def _sc_body(feat_hbm, ky_hbm, kx_hbm, out_hbm,
             f0, f1, f2, f3, y0, y1, y2, y3, x0, x1, x2, x3,
             c0, c1, c2, c3, ones, acc, cnt,
             l0, l1, l2, l3, s0, s1, s2, s3):
    c = lax.axis_index("c")
    s = lax.axis_index("s")
    feat_bufs = [f0, f1, f2, f3]
    y_bufs = [y0, y1, y2, y3]
    x_bufs = [x0, x1, x2, x3]
    cell_bufs = [c0, c1, c2, c3]
    load_sems = [l0, l1, l2, l3]
    scat_sems = [s0, s1, s2, s3]

    zeros16f = jnp.zeros((16,), jnp.float32)
    ones16f = jnp.ones((16,), jnp.float32)

    lb = s // 2
    b = c * 8 + lb
    half = s % 2
    n0 = half * NODES_PER_TILE
    row_base = lb * HW

    def _zero_row(i, _):
        for j in range(D // 16):
            f0[i, pl.ds(j * 16, 16)] = zeros16f
            ones[i, pl.ds(j * 16, 16)] = ones16f
        return 0
    lax.fori_loop(0, CHUNK, _zero_row, 0)

    pltpu.sync_copy(f0.at[pl.ds(0, ROWS_PER_TILE)],
                    acc.at[pl.ds(s * ROWS_PER_TILE, ROWS_PER_TILE)])
    pltpu.sync_copy(f0.at[pl.ds(0, ROWS_PER_TILE)],
                    cnt.at[pl.ds(s * ROWS_PER_TILE, ROWS_PER_TILE)])
    plsc.subcore_barrier()

    def start_load(i, k):
        pltpu.async_copy(feat_hbm.at[b, pl.ds(n0 + i * CHUNK, CHUNK)],
                         feat_bufs[k], load_sems[k])
        pltpu.async_copy(ky_hbm.at[b, pl.ds(n0 + i * CHUNK, CHUNK)],
                         y_bufs[k], load_sems[k])
        pltpu.async_copy(kx_hbm.at[b, pl.ds(n0 + i * CHUNK, CHUNK)],
                         x_bufs[k], load_sems[k])

    def wait_load(k):
        pltpu.make_async_copy(feat_hbm.at[b, pl.ds(n0, CHUNK)],
                              feat_bufs[k], load_sems[k]).wait()
        pltpu.make_async_copy(ky_hbm.at[b, pl.ds(n0, CHUNK)],
                              y_bufs[k], load_sems[k]).wait()
        pltpu.make_async_copy(kx_hbm.at[b, pl.ds(n0, CHUNK)],
                              x_bufs[k], load_sems[k]).wait()

    def compute_cells(i, k):
        cb = cell_bufs[k]
        for j in range(CHUNK // 16):
            yv = y_bufs[k][pl.ds(j * 16, 16)]
            xv = x_bufs[k][pl.ds(j * 16, 16)]
            cb[pl.ds(j * 16, 16)] = yv * 16 + xv + row_base

    def start_scats(k):
        pltpu.async_copy(feat_bufs[k], acc.at[cell_bufs[k]], scat_sems[k],
                         add=True)
        pltpu.async_copy(ones, cnt.at[cell_bufs[k]], scat_sems[k], add=True)

    def wait_scats(k):
        pltpu.make_async_copy(feat_bufs[k], acc.at[cell_bufs[k]],
                              scat_sems[k]).wait()
        pltpu.make_async_copy(ones, cnt.at[cell_bufs[k]],
                              scat_sems[k]).wait()

    start_load(0, 0)
    start_load(1, 1)
    wait_load(0)
    compute_cells(0, 0)
    start_scats(0)
    start_load(2, 2)
    wait_load(1)
    compute_cells(1, 1)
    start_scats(1)
    start_load(3, 3)

    def _round(m, _):
        i0 = 2 + m * 4
        for j in range(4):
            i = i0 + j
            k = (2 + j) % 4
            wait_load(k)
            compute_cells(i, k)
            start_scats(k)
            wait_scats((k + 2) % 4)
            start_load(i + 2, (k + 2) % 4)
        return 0
    lax.fori_loop(0, (NCHUNK - 4) // 4, _round, 0)

    wait_load(2)
    compute_cells(NCHUNK - 2, 2)
    start_scats(2)
    wait_scats(0)
    wait_load(3)
    compute_cells(NCHUNK - 1, 3)
    start_scats(3)
    wait_scats(1)
    wait_scats(2)
    wait_scats(3)
    plsc.subcore_barrier()

    rows0 = s * ROWS_PER_TILE
    pltpu.sync_copy(acc.at[pl.ds(rows0, ROWS_PER_TILE)],
                    f0.at[pl.ds(0, ROWS_PER_TILE)])
    pltpu.sync_copy(cnt.at[pl.ds(rows0, ROWS_PER_TILE)],
                    f1.at[pl.ds(0, ROWS_PER_TILE)])

    def _cell(i, _):
        inv = 1.0 / jnp.maximum(f1[i, pl.ds(0, 16)], 1.0)
        for j in range(D // 16):
            f0[i, pl.ds(j * 16, 16)] = f0[i, pl.ds(j * 16, 16)] * inv
        return 0
    lax.fori_loop(0, ROWS_PER_TILE, _cell, 0)

    cell0 = half * ROWS_PER_TILE
    pltpu.sync_copy(f0.at[pl.ds(0, ROWS_PER_TILE)],
                    out_hbm.at[b, pl.ds(cell0, ROWS_PER_TILE)])


@jax.jit
def _graph_to_cnn(features, ky, kx):
    mesh = plsc.VectorSubcoreMesh(core_axis_name="c", subcore_axis_name="s")
    run = functools.partial(
        pl.kernel,
        out_type=jax.ShapeDtypeStruct((B, HW, D), jnp.float32),
        mesh=mesh,
        scratch_types=[
        ] + [pltpu.VMEM((CHUNK, D), jnp.float32)] * NBUF
          + [pltpu.VMEM((CHUNK,), jnp.int32)] * NBUF * 2
          + [pltpu.VMEM((CHUNK,), jnp.int32)] * NBUF
          + [
            pltpu.VMEM((CHUNK, D), jnp.float32),
            pltpu.VMEM_SHARED((ROWS_PER_SC, D), jnp.float32),
            pltpu.VMEM_SHARED((ROWS_PER_SC, D), jnp.float32),
        ] + [pltpu.SemaphoreType.DMA] * (2 * NBUF),
    )(_sc_body)
    return run(features, ky, kx)


def kernel(features, key_locs):
    kl = key_locs.astype(jnp.int32)
    ky = kl[..., 0]
    kx = kl[..., 1]
    out = _graph_to_cnn(features, ky, kx)
    return jnp.transpose(out, (0, 2, 1)).reshape(B, D, 16, 16)


# speedup vs baseline: 9.7136x; 1.3515x over previous
import functools

import jax
import jax.numpy as jnp
from jax import lax
from jax.experimental import pallas as pl
from jax.experimental.pallas import tpu as pltpu
from jax.experimental.pallas import tpu_sc as plsc

B = 16
N = 16384
D = 128
HW = 256

NODES_PER_TILE = N // 2
CHUNK = 64
NCHUNK = NODES_PER_TILE // CHUNK
ROWS_PER_SC = 8 * HW
ROWS_PER_TILE = ROWS_PER_SC // 16
NBUF = 4


def _sc_body(feat_hbm, ky_hbm, kx_hbm, out_hbm,
             f0, f1, f2, f3, y0, y1, y2, y3, x0, x1, x2, x3,
             c0, c1, c2, c3, hist_a, hist_b, acc, cnt,
             l0, l1, l2, l3, s0, s1, s2, s3):
    c = lax.axis_index("c")
    s = lax.axis_index("s")
    feat_bufs = [f0, f1, f2, f3]
    y_bufs = [y0, y1, y2, y3]
    x_bufs = [x0, x1, x2, x3]
    cell_bufs = [c0, c1, c2, c3]
    load_sems = [l0, l1, l2, l3]
    scat_sems = [s0, s1, s2, s3]

    zeros16f = jnp.zeros((16,), jnp.float32)
    ones16f = jnp.ones((16,), jnp.float32)

    lb = s // 2
    b = c * 8 + lb
    half = s % 2
    n0 = half * NODES_PER_TILE
    row_base = lb * HW

    def _zero_row(i, _):
        for j in range(D // 16):
            f0[i, pl.ds(j * 16, 16)] = zeros16f
        return 0
    lax.fori_loop(0, CHUNK, _zero_row, 0)

    def _zero_hist(i, _):
        hist_a[pl.ds(i * 16, 16)] = zeros16f
        hist_b[pl.ds(i * 16, 16)] = zeros16f
        return 0
    lax.fori_loop(0, HW, _zero_hist, 0)

    for h in range(ROWS_PER_TILE // CHUNK):
        pltpu.sync_copy(f0, acc.at[pl.ds(s * ROWS_PER_TILE + h * CHUNK, CHUNK)])
        pltpu.sync_copy(f0, cnt.at[pl.ds(s * ROWS_PER_TILE + h * CHUNK, CHUNK)])
    plsc.subcore_barrier()

    def start_load(i, k):
        pltpu.async_copy(feat_hbm.at[b, pl.ds(n0 + i * CHUNK, CHUNK)],
                         feat_bufs[k], load_sems[k])
        pltpu.async_copy(ky_hbm.at[b, pl.ds(n0 + i * CHUNK, CHUNK)],
                         y_bufs[k], load_sems[k])
        pltpu.async_copy(kx_hbm.at[b, pl.ds(n0 + i * CHUNK, CHUNK)],
                         x_bufs[k], load_sems[k])

    def wait_load(k):
        pltpu.make_async_copy(feat_hbm.at[b, pl.ds(n0, CHUNK)],
                              feat_bufs[k], load_sems[k]).wait()
        pltpu.make_async_copy(ky_hbm.at[b, pl.ds(n0, CHUNK)],
                              y_bufs[k], load_sems[k]).wait()
        pltpu.make_async_copy(kx_hbm.at[b, pl.ds(n0, CHUNK)],
                              x_bufs[k], load_sems[k]).wait()

    def compute_cells(i, k):
        cb = cell_bufs[k]
        for j in range(CHUNK // 16):
            yv = y_bufs[k][pl.ds(j * 16, 16)]
            xv = x_bufs[k][pl.ds(j * 16, 16)]
            cb[pl.ds(j * 16, 16)] = yv * 16 + xv + row_base

    def start_scats(k):
        pltpu.async_copy(feat_bufs[k], acc.at[cell_bufs[k]], scat_sems[k],
                         add=True)

    def wait_scats(k):
        pltpu.make_async_copy(feat_bufs[k], acc.at[cell_bufs[k]],
                              scat_sems[k]).wait()

    def update_hist(k):
        cb = cell_bufs[k]
        def _upd(j, _):
            cv = (cb[pl.ds(j * 16, 16)] - row_base) * 16
            for l in range(0, 16, 2):
                ca = cv[l]
                cbv = cv[l + 1]
                hist_a[pl.ds(ca, 16)] = hist_a[pl.ds(ca, 16)] + ones16f
                hist_b[pl.ds(cbv, 16)] = hist_b[pl.ds(cbv, 16)] + ones16f
            return 0
        lax.fori_loop(0, CHUNK // 16, _upd, 0)

    start_load(0, 0)
    start_load(1, 1)
    wait_load(0)
    compute_cells(0, 0)
    start_scats(0)
    start_load(2, 2)
    update_hist(0)
    wait_load(1)
    compute_cells(1, 1)
    start_scats(1)
    start_load(3, 3)
    update_hist(1)

    def _round(m, _):
        i0 = 2 + m * 4
        for j in range(4):
            i = i0 + j
            k = (2 + j) % 4
            wait_load(k)
            compute_cells(i, k)
            start_scats(k)
            wait_scats((k + 2) % 4)
            start_load(i + 2, (k + 2) % 4)
            update_hist(k)
        return 0
    lax.fori_loop(0, (NCHUNK - 4) // 4, _round, 0)

    wait_load(2)
    compute_cells(NCHUNK - 2, 2)
    start_scats(2)
    wait_scats(0)
    update_hist(2)
    wait_load(3)
    compute_cells(NCHUNK - 1, 3)
    start_scats(3)
    wait_scats(1)
    update_hist(3)

    wait_scats(2)
    wait_scats(3)
    i0_16 = lax.iota(jnp.int32, 16)
    NG = HW // CHUNK

    for g in range(NG):
        fb = feat_bufs[g]
        cb = cell_bufs[g]

        def _bld(r, _, g=g, fb=fb):
            hv = (hist_a[pl.ds((g * CHUNK + r) * 16, 16)]
                  + hist_b[pl.ds((g * CHUNK + r) * 16, 16)])
            for j in range(D // 16):
                fb[r, pl.ds(j * 16, 16)] = hv
            return 0
        lax.fori_loop(0, CHUNK, _bld, 0)
        for j in range(CHUNK // 16):
            cb[pl.ds(j * 16, 16)] = i0_16 + (row_base + g * CHUNK + j * 16)
        pltpu.async_copy(fb, cnt.at[cb], scat_sems[g], add=True)

    for g in range(NG):
        pltpu.make_async_copy(feat_bufs[g], cnt.at[cell_bufs[g]],
                              scat_sems[g]).wait()
    plsc.subcore_barrier()

    rows0 = s * ROWS_PER_TILE
    cell0 = half * ROWS_PER_TILE
    for h in range(ROWS_PER_TILE // CHUNK):
        pltpu.sync_copy(acc.at[pl.ds(rows0 + h * CHUNK, CHUNK)], f0)
        pltpu.sync_copy(cnt.at[pl.ds(rows0 + h * CHUNK, CHUNK)], f1)

        def _cell(i, _):
            inv = 1.0 / jnp.maximum(f1[i, pl.ds(0, 16)], 1.0)
            for j in range(D // 16):
                f0[i, pl.ds(j * 16, 16)] = f0[i, pl.ds(j * 16, 16)] * inv
            return 0
        lax.fori_loop(0, CHUNK, _cell, 0)
        pltpu.sync_copy(f0, out_hbm.at[b, pl.ds(cell0 + h * CHUNK, CHUNK)])


@jax.jit
def _graph_to_cnn(features, ky, kx):
    mesh = plsc.VectorSubcoreMesh(core_axis_name="c", subcore_axis_name="s")
    run = functools.partial(
        pl.kernel,
        out_type=jax.ShapeDtypeStruct((B, HW, D), jnp.float32),
        mesh=mesh,
        scratch_types=[
        ] + [pltpu.VMEM((CHUNK, D), jnp.float32)] * NBUF
          + [pltpu.VMEM((CHUNK,), jnp.int32)] * NBUF * 2
          + [pltpu.VMEM((CHUNK,), jnp.int32)] * NBUF
          + [
            pltpu.VMEM((HW * 16,), jnp.float32),
            pltpu.VMEM((HW * 16,), jnp.float32),
            pltpu.VMEM_SHARED((ROWS_PER_SC, D), jnp.float32),
            pltpu.VMEM_SHARED((ROWS_PER_SC, D), jnp.float32),
        ] + [pltpu.SemaphoreType.DMA] * (2 * NBUF),
    )(_sc_body)
    return run(features, ky, kx)


def kernel(features, key_locs):
    kl = key_locs.astype(jnp.int32)
    ky = kl[..., 0]
    kx = kl[..., 1]
    out = _graph_to_cnn(features, ky, kx)
    return jnp.transpose(out, (0, 2, 1)).reshape(B, D, 16, 16)


# speedup vs baseline: 10.1199x; 1.0418x over previous
import functools

import jax
import jax.numpy as jnp
from jax import lax
from jax.experimental import pallas as pl
from jax.experimental.pallas import tpu as pltpu
from jax.experimental.pallas import tpu_sc as plsc

B = 16
N = 16384
D = 128
HW = 256

NODES_PER_TILE = N // 2
CHUNK = 128
NCHUNK = NODES_PER_TILE // CHUNK
ROWS_PER_SC = 8 * HW
ROWS_PER_TILE = ROWS_PER_SC // 16
NBUF = 4


def _sc_body(feat_hbm, ky_hbm, kx_hbm, out_hbm,
             f0, f1, f2, f3, y0, y1, y2, y3, x0, x1, x2, x3,
             c0, c1, c2, c3, hist_a, hist_b, acc, cnt,
             l0, l1, l2, l3, s0, s1, s2, s3):
    c = lax.axis_index("c")
    s = lax.axis_index("s")
    feat_bufs = [f0, f1, f2, f3]
    y_bufs = [y0, y1, y2, y3]
    x_bufs = [x0, x1, x2, x3]
    cell_bufs = [c0, c1, c2, c3]
    load_sems = [l0, l1, l2, l3]
    scat_sems = [s0, s1, s2, s3]

    zeros16f = jnp.zeros((16,), jnp.float32)
    ones16f = jnp.ones((16,), jnp.float32)

    lb = s // 2
    b = c * 8 + lb
    half = s % 2
    n0 = half * NODES_PER_TILE
    row_base = lb * HW

    def _zero_row(i, _):
        for j in range(D // 16):
            f0[i, pl.ds(j * 16, 16)] = zeros16f
        return 0
    lax.fori_loop(0, CHUNK, _zero_row, 0)

    def _zero_hist(i, _):
        hist_a[pl.ds(i * 16, 16)] = zeros16f
        hist_b[pl.ds(i * 16, 16)] = zeros16f
        return 0
    lax.fori_loop(0, HW, _zero_hist, 0)

    for h in range(ROWS_PER_TILE // CHUNK):
        pltpu.sync_copy(f0, acc.at[pl.ds(s * ROWS_PER_TILE + h * CHUNK, CHUNK)])
        pltpu.sync_copy(f0, cnt.at[pl.ds(s * ROWS_PER_TILE + h * CHUNK, CHUNK)])
    plsc.subcore_barrier()

    def start_load(i, k):
        pltpu.async_copy(feat_hbm.at[b, pl.ds(n0 + i * CHUNK, CHUNK)],
                         feat_bufs[k], load_sems[k])
        pltpu.async_copy(ky_hbm.at[b, pl.ds(n0 + i * CHUNK, CHUNK)],
                         y_bufs[k], load_sems[k])
        pltpu.async_copy(kx_hbm.at[b, pl.ds(n0 + i * CHUNK, CHUNK)],
                         x_bufs[k], load_sems[k])

    def wait_load(k):
        pltpu.make_async_copy(feat_hbm.at[b, pl.ds(n0, CHUNK)],
                              feat_bufs[k], load_sems[k]).wait()
        pltpu.make_async_copy(ky_hbm.at[b, pl.ds(n0, CHUNK)],
                              y_bufs[k], load_sems[k]).wait()
        pltpu.make_async_copy(kx_hbm.at[b, pl.ds(n0, CHUNK)],
                              x_bufs[k], load_sems[k]).wait()

    def compute_cells(i, k):
        cb = cell_bufs[k]
        for j in range(CHUNK // 16):
            yv = y_bufs[k][pl.ds(j * 16, 16)]
            xv = x_bufs[k][pl.ds(j * 16, 16)]
            cb[pl.ds(j * 16, 16)] = yv * 16 + xv + row_base

    def start_scats(k):
        pltpu.async_copy(feat_bufs[k], acc.at[cell_bufs[k]], scat_sems[k],
                         add=True)

    def wait_scats(k):
        pltpu.make_async_copy(feat_bufs[k], acc.at[cell_bufs[k]],
                              scat_sems[k]).wait()

    def update_hist(k):
        cb = cell_bufs[k]
        def _upd(j, _):
            cv = (cb[pl.ds(j * 16, 16)] - row_base) * 16
            for l in range(0, 16, 2):
                ca = cv[l]
                cbv = cv[l + 1]
                hist_a[pl.ds(ca, 16)] = hist_a[pl.ds(ca, 16)] + ones16f
                hist_b[pl.ds(cbv, 16)] = hist_b[pl.ds(cbv, 16)] + ones16f
            return 0
        lax.fori_loop(0, CHUNK // 16, _upd, 0)

    start_load(0, 0)
    start_load(1, 1)
    wait_load(0)
    compute_cells(0, 0)
    start_scats(0)
    start_load(2, 2)
    update_hist(0)
    wait_load(1)
    compute_cells(1, 1)
    start_scats(1)
    start_load(3, 3)
    update_hist(1)

    def _round(m, _):
        i0 = 2 + m * 4
        for j in range(4):
            i = i0 + j
            k = (2 + j) % 4
            wait_load(k)
            compute_cells(i, k)
            start_scats(k)
            wait_scats((k + 2) % 4)
            start_load(i + 2, (k + 2) % 4)
            update_hist(k)
        return 0
    lax.fori_loop(0, (NCHUNK - 4) // 4, _round, 0)

    wait_load(2)
    compute_cells(NCHUNK - 2, 2)
    start_scats(2)
    wait_scats(0)
    update_hist(2)
    wait_load(3)
    compute_cells(NCHUNK - 1, 3)
    start_scats(3)
    wait_scats(1)
    update_hist(3)

    wait_scats(2)
    wait_scats(3)
    i0_16 = lax.iota(jnp.int32, 16)
    NG = HW // CHUNK

    for g in range(NG):
        fb = feat_bufs[g]
        cb = cell_bufs[g]

        def _bld(r, _, g=g, fb=fb):
            hv = (hist_a[pl.ds((g * CHUNK + r) * 16, 16)]
                  + hist_b[pl.ds((g * CHUNK + r) * 16, 16)])
            for j in range(D // 16):
                fb[r, pl.ds(j * 16, 16)] = hv
            return 0
        lax.fori_loop(0, CHUNK, _bld, 0)
        for j in range(CHUNK // 16):
            cb[pl.ds(j * 16, 16)] = i0_16 + (row_base + g * CHUNK + j * 16)
        pltpu.async_copy(fb, cnt.at[cb], scat_sems[g], add=True)

    for g in range(NG):
        pltpu.make_async_copy(feat_bufs[g], cnt.at[cell_bufs[g]],
                              scat_sems[g]).wait()
    plsc.subcore_barrier()

    rows0 = s * ROWS_PER_TILE
    cell0 = half * ROWS_PER_TILE
    for h in range(ROWS_PER_TILE // CHUNK):
        pltpu.sync_copy(acc.at[pl.ds(rows0 + h * CHUNK, CHUNK)], f0)
        pltpu.sync_copy(cnt.at[pl.ds(rows0 + h * CHUNK, CHUNK)], f1)

        def _cell(i, _):
            inv = 1.0 / jnp.maximum(f1[i, pl.ds(0, 16)], 1.0)
            for j in range(D // 16):
                f0[i, pl.ds(j * 16, 16)] = f0[i, pl.ds(j * 16, 16)] * inv
            return 0
        lax.fori_loop(0, CHUNK, _cell, 0)
        pltpu.sync_copy(f0, out_hbm.at[b, pl.ds(cell0 + h * CHUNK, CHUNK)])


@jax.jit
def _graph_to_cnn(features, ky, kx):
    mesh = plsc.VectorSubcoreMesh(core_axis_name="c", subcore_axis_name="s")
    run = functools.partial(
        pl.kernel,
        out_type=jax.ShapeDtypeStruct((B, HW, D), jnp.float32),
        mesh=mesh,
        scratch_types=[
        ] + [pltpu.VMEM((CHUNK, D), jnp.float32)] * NBUF
          + [pltpu.VMEM((CHUNK,), jnp.int32)] * NBUF * 2
          + [pltpu.VMEM((CHUNK,), jnp.int32)] * NBUF
          + [
            pltpu.VMEM((HW * 16,), jnp.float32),
            pltpu.VMEM((HW * 16,), jnp.float32),
            pltpu.VMEM_SHARED((ROWS_PER_SC, D), jnp.float32),
            pltpu.VMEM_SHARED((ROWS_PER_SC, D), jnp.float32),
        ] + [pltpu.SemaphoreType.DMA] * (2 * NBUF),
    )(_sc_body)
    return run(features, ky, kx)


def kernel(features, key_locs):
    kl = key_locs.astype(jnp.int32)
    ky = kl[..., 0]
    kx = kl[..., 1]
    out = _graph_to_cnn(features, ky, kx)
    return jnp.transpose(out, (0, 2, 1)).reshape(B, D, 16, 16)
